# baseline (speedup 1.0000x reference)
import functools

import jax
import jax.numpy as jnp
from jax import lax
from jax.experimental import pallas as pl
from jax.experimental.pallas import tpu as pltpu
from jax.experimental.pallas import tpu_sc as plsc

N = 10000
E = 320000
F_IN = 128
H = 64
C = 64
R = 4

NS = 16
NC = 2
RN = R * N
TROWS = RN + 16
JUNK_DST = N
KCH = 256
SLOTS = 327680
NCHUNK = SLOTS // KCH
ROWS_PER_TILE = 2560
CW = 8
HW = 32

_i32 = jnp.int32
_f32 = jnp.float32


def _pack_yt(ms):
    return jnp.concatenate(
        [ms[r][:, c * HW:(c + 1) * HW] for c in range(NC) for r in range(R)],
        axis=-1,
    )


def _tc_pre_body(x_ref, wrel_ref, wroot_ref, b_ref, yt_ref, root_ref):
    xb = x_ref[...]
    ms = [jnp.dot(xb, wrel_ref[r], preferred_element_type=_f32)
          for r in range(R)]
    yt_ref[...] = _pack_yt(ms)
    root_ref[...] = (
        jnp.dot(xb, wroot_ref[...], preferred_element_type=_f32) + b_ref[...]
    )


def _tc_pre(x, w_rel, w_root, b):
    f = x.shape[1]
    return pl.pallas_call(
        _tc_pre_body,
        grid=(10,),
        in_specs=[
            pl.BlockSpec((1000, f), lambda i: (i, 0)),
            pl.BlockSpec((R, f, H), lambda i: (0, 0, 0)),
            pl.BlockSpec((f, H), lambda i: (0, 0)),
            pl.BlockSpec((1, H), lambda i: (0, 0)),
        ],
        out_specs=[
            pl.BlockSpec((1000, NC * R * HW), lambda i: (i, 0)),
            pl.BlockSpec((1000, H), lambda i: (i, 0)),
        ],
        out_shape=[
            jax.ShapeDtypeStruct((N, NC * R * HW), _f32),
            jax.ShapeDtypeStruct((N, H), _f32),
        ],
    )(x, w_rel, w_root, b.reshape(1, H))


def _s_full(s_ref, r):
    return jnp.concatenate(
        [s_ref[c][:, r * HW:(r + 1) * HW] for c in range(NC)], axis=-1)


def _tc_mid_body(root1_ref, s_ref, cnt_ref, wrel_ref, wroot_ref, b_ref,
                 emb_ref, yt_ref, root2_ref):
    cval = cnt_ref[...]
    acc = root1_ref[...]
    for r in range(R):
        d = jnp.maximum(cval[:, r:r + 1] + cval[:, R + r:R + r + 1], 1.0)
        acc = acc + _s_full(s_ref, r) / d
    emb = jnp.maximum(acc, 0.0)
    emb_ref[...] = emb
    ms = [jnp.dot(emb, wrel_ref[r], preferred_element_type=_f32)
          for r in range(R)]
    yt_ref[...] = _pack_yt(ms)
    root2_ref[...] = (
        jnp.dot(emb, wroot_ref[...], preferred_element_type=_f32) + b_ref[...]
    )


def _tc_mid(root1, s_part, cnt8, w_rel, w_root, b):
    return pl.pallas_call(
        _tc_mid_body,
        grid=(10,),
        in_specs=[
            pl.BlockSpec((1000, H), lambda i: (i, 0)),
            pl.BlockSpec((NC, 1000, R * HW), lambda i: (0, i, 0)),
            pl.BlockSpec((1000, 2 * R), lambda i: (i, 0)),
            pl.BlockSpec((R, H, C), lambda i: (0, 0, 0)),
            pl.BlockSpec((H, C), lambda i: (0, 0)),
            pl.BlockSpec((1, C), lambda i: (0, 0)),
        ],
        out_specs=[
            pl.BlockSpec((1000, H), lambda i: (i, 0)),
            pl.BlockSpec((1000, NC * R * HW), lambda i: (i, 0)),
            pl.BlockSpec((1000, C), lambda i: (i, 0)),
        ],
        out_shape=[
            jax.ShapeDtypeStruct((N, H), _f32),
            jax.ShapeDtypeStruct((N, NC * R * HW), _f32),
            jax.ShapeDtypeStruct((N, C), _f32),
        ],
    )(root1, s_part, cnt8, w_rel, w_root, b.reshape(1, C))


def _tc_post_body(root2_ref, s_ref, cnt_ref, out_ref):
    cval = cnt_ref[...]
    logits = root2_ref[...]
    for r in range(R):
        d = jnp.maximum(cval[:, r:r + 1] + cval[:, R + r:R + r + 1], 1.0)
        logits = logits + _s_full(s_ref, r) / d
    m = jnp.max(logits, axis=1, keepdims=True)
    sh = logits - m
    out_ref[...] = sh - jnp.log(jnp.sum(jnp.exp(sh), axis=1, keepdims=True))


def _tc_post(root2, s_part, cnt8):
    return pl.pallas_call(
        _tc_post_body,
        grid=(10,),
        in_specs=[
            pl.BlockSpec((1000, C), lambda i: (i, 0)),
            pl.BlockSpec((NC, 1000, R * HW), lambda i: (0, i, 0)),
            pl.BlockSpec((1000, 2 * R), lambda i: (i, 0)),
        ],
        out_specs=pl.BlockSpec((1000, C), lambda i: (i, 0)),
        out_shape=jax.ShapeDtypeStruct((N, C), _f32),
    )(root2, s_part, cnt8)


_SC_MESH = plsc.VectorSubcoreMesh(core_axis_name="c", subcore_axis_name="s")
_SC_PARAMS = pltpu.CompilerParams(use_tc_tiling_on_sc=False)

_CPT = NCHUNK // NS
_FULL_T = RN // ROWS_PER_TILE
_TAIL = RN - _FULL_T * ROWS_PER_TILE


def _sc_epilogue(cid, sid, tab, out_hbm):
    base = sid * ROWS_PER_TILE

    @pl.when(sid < _FULL_T)
    def _full():
        pltpu.sync_copy(
            tab.at[pl.ds(base, ROWS_PER_TILE)],
            out_hbm.at[cid, pl.ds(base, ROWS_PER_TILE)],
        )

    @pl.when(sid == _FULL_T)
    def _tail():
        pltpu.sync_copy(
            tab.at[pl.ds(_FULL_T * ROWS_PER_TILE, _TAIL)],
            out_hbm.at[cid, pl.ds(_FULL_T * ROWS_PER_TILE, _TAIL)],
        )


def _sc_fill(sid, src_hbm, tab):
    base = sid * ROWS_PER_TILE

    @pl.when(sid < _FULL_T)
    def _full():
        pltpu.sync_copy(src_hbm, tab.at[pl.ds(base, ROWS_PER_TILE)])

    @pl.when(sid == _FULL_T)
    def _tail():
        pltpu.sync_copy(
            src_hbm.at[pl.ds(0, _TAIL)],
            tab.at[pl.ds(_FULL_T * ROWS_PER_TILE, _TAIL)],
        )


def _agg_body(with_count, yt_hbm, epack_hbm, zt_hbm, zc_hbm, ones_hbm,
              s_out, cnt_out, ebuf, gbuf, dbuf, rows, stab,
              gsem, esem, ssem, onesb=None, ctab=None):
    cid = lax.axis_index("c")
    sid = lax.axis_index("s")
    coff = cid * R
    row0 = sid * _CPT

    def mk_idx(b):
        for v in range(KCH // 16):
            s = ebuf[b, pl.ds(v * 16, 16)]
            d = ebuf[b, pl.ds(KCH + v * 16, 16)]
            e = ebuf[b, pl.ds(2 * KCH + v * 16, 16)]
            gbuf[b, pl.ds(v * 16, 16)] = s * (NC * R) + e + coff
            dbuf[b, pl.ds(v * 16, 16)] = d * R + e

    _sc_fill(sid, zt_hbm, stab)
    if with_count:
        do_count = jnp.logical_or(
            jnp.logical_and(cid == 0, sid < NS // 2),
            jnp.logical_and(cid == 1, sid >= NS // 2),
        )
        _sc_fill(sid, zc_hbm, ctab)
        pltpu.sync_copy(ones_hbm, onesb)
    plsc.subcore_barrier()

    pltpu.sync_copy(epack_hbm.at[row0], ebuf.at[0])
    mk_idx(0)
    pltpu.async_copy(yt_hbm.at[gbuf.at[0]], rows.at[0], gsem)
    pltpu.async_copy(epack_hbm.at[row0 + 1], ebuf.at[1], esem)

    def pair(jj, carry):
        for b in (0, 1):
            j = jj * 2 + b
            nb = 1 - b

            @pl.when(j < _CPT - 1)
            def _prep():
                pltpu.make_async_copy(
                    epack_hbm.at[row0 + j + 1], ebuf.at[nb], esem).wait()
                mk_idx(nb)

            pltpu.make_async_copy(
                yt_hbm.at[gbuf.at[b]], rows.at[b], gsem).wait()

            @pl.when(j > 0)
            def _drain():
                pltpu.make_async_copy(
                    rows.at[nb], stab.at[dbuf.at[nb]], ssem).wait()

            @pl.when(j < _CPT - 1)
            def _gather():
                pltpu.async_copy(yt_hbm.at[gbuf.at[nb]], rows.at[nb], gsem)

            @pl.when(j < _CPT - 2)
            def _edges():
                pltpu.async_copy(epack_hbm.at[row0 + j + 2], ebuf.at[b], esem)

            pltpu.async_copy(rows.at[b], stab.at[dbuf.at[b]], ssem, add=True)
            if with_count:
                @pl.when(do_count)
                def _count():
                    pltpu.sync_copy(onesb, ctab.at[dbuf.at[b]], add=True)
        return carry

    lax.fori_loop(0, _CPT // 2, pair, 0)
    pltpu.make_async_copy(
        rows.at[(_CPT - 1) % 2], stab.at[dbuf.at[(_CPT - 1) % 2]], ssem).wait()
    plsc.subcore_barrier()
    _sc_epilogue(cid, sid, stab, s_out)
    if with_count:
        _sc_epilogue(cid, sid, ctab, cnt_out)


_AGG_SCRATCH = [
    pltpu.VMEM((2, 3 * KCH), _i32),
    pltpu.VMEM((2, KCH), _i32),
    pltpu.VMEM((2, KCH), _i32),
    pltpu.VMEM((2, KCH, HW), _f32),
    pltpu.VMEM_SHARED((TROWS, HW), _f32),
    pltpu.SemaphoreType.DMA,
    pltpu.SemaphoreType.DMA,
    pltpu.SemaphoreType.DMA,
]


@functools.partial(
    pl.kernel,
    out_type=[
        jax.ShapeDtypeStruct((NC, RN, HW), _f32),
        jax.ShapeDtypeStruct((NC, RN, CW), _f32),
    ],
    mesh=_SC_MESH,
    compiler_params=_SC_PARAMS,
    scratch_types=_AGG_SCRATCH + [
        pltpu.VMEM((KCH, CW), _f32),
        pltpu.VMEM_SHARED((TROWS, CW), _f32),
    ],
)
def _sc_agg_count(yt_hbm, epack_hbm, zt_hbm, zc_hbm, ones_hbm,
                  s_out, cnt_out, ebuf, gbuf, dbuf, rows, stab,
                  gsem, esem, ssem, onesb, ctab):
    _agg_body(True, yt_hbm, epack_hbm, zt_hbm, zc_hbm, ones_hbm,
              s_out, cnt_out, ebuf, gbuf, dbuf, rows, stab,
              gsem, esem, ssem, onesb=onesb, ctab=ctab)


@functools.partial(
    pl.kernel,
    out_type=jax.ShapeDtypeStruct((NC, RN, HW), _f32),
    mesh=_SC_MESH,
    compiler_params=_SC_PARAMS,
    scratch_types=_AGG_SCRATCH,
)
def _sc_agg(yt_hbm, epack_hbm, zt_hbm, zc_hbm, ones_hbm, s_out, *scratch):
    _agg_body(False, yt_hbm, epack_hbm, zt_hbm, zc_hbm, ones_hbm,
              s_out, None, *scratch)


def kernel(x, edge_index, edge_type, W1_rel, W1_root, b1, W2_rel, W2_root, b2):
    src = edge_index[0].astype(_i32)
    dst = edge_index[1].astype(_i32)
    et = edge_type.astype(_i32)
    pad = SLOTS - E
    srcp = jnp.concatenate([src, jnp.zeros((pad,), _i32)])
    dstp = jnp.concatenate([dst, jnp.full((pad,), JUNK_DST, _i32)])
    etp = jnp.concatenate([et, jnp.zeros((pad,), _i32)])
    epack = jnp.stack(
        [srcp.reshape(NCHUNK, KCH),
         dstp.reshape(NCHUNK, KCH),
         etp.reshape(NCHUNK, KCH)], axis=1,
    ).reshape(NCHUNK, 3 * KCH)

    zt = jnp.zeros((ROWS_PER_TILE, HW), _f32)
    zc = jnp.zeros((ROWS_PER_TILE, CW), _f32)
    ones = jnp.ones((KCH, CW), _f32)

    yt1, root1 = _tc_pre(x, W1_rel, W1_root, b1)
    s1, cnt_part = _sc_agg_count(
        yt1.reshape(NC * RN, HW), epack, zt, zc, ones)
    cnt8 = jnp.swapaxes(
        cnt_part[..., 0].reshape(NC, N, R), 0, 1).reshape(N, NC * R)
    s1v = s1.reshape(NC, N, R * HW)

    emb, yt2, root2 = _tc_mid(root1, s1v, cnt8, W2_rel, W2_root, b2)
    s2 = _sc_agg(yt2.reshape(NC * RN, HW), epack, zt, zc, ones)
    s2v = s2.reshape(NC, N, R * HW)

    logsm = _tc_post(root2, s2v, cnt8)
    return (logsm, emb)


# speedup vs baseline: 1.0723x; 1.0723x over previous
import functools

import jax
import jax.numpy as jnp
from jax import lax
from jax.experimental import pallas as pl
from jax.experimental.pallas import tpu as pltpu
from jax.experimental.pallas import tpu_sc as plsc

N = 10000
E = 320000
F_IN = 128
H = 64
C = 64
R = 4

NS = 16
NC = 2
RN = R * N
TROWS = RN + 16
JUNK_DST = N
KCH = 256
SLOTS = 327680
NCHUNK = SLOTS // KCH
ROWS_PER_TILE = 2560
CW = 8
HW = 32

_i32 = jnp.int32
_f32 = jnp.float32


def _pack_yt(ms):
    return jnp.concatenate(
        [ms[r][:, c * HW:(c + 1) * HW] for c in range(NC) for r in range(R)],
        axis=-1,
    )


def _tc_pre_body(x_ref, wrel_ref, wroot_ref, b_ref, yt_ref, root_ref):
    xb = x_ref[...]
    ms = [jnp.dot(xb, wrel_ref[r], preferred_element_type=_f32)
          for r in range(R)]
    yt_ref[...] = _pack_yt(ms)
    root_ref[...] = (
        jnp.dot(xb, wroot_ref[...], preferred_element_type=_f32) + b_ref[...]
    )


def _tc_pre(x, w_rel, w_root, b):
    f = x.shape[1]
    return pl.pallas_call(
        _tc_pre_body,
        grid=(10,),
        in_specs=[
            pl.BlockSpec((1000, f), lambda i: (i, 0)),
            pl.BlockSpec((R, f, H), lambda i: (0, 0, 0)),
            pl.BlockSpec((f, H), lambda i: (0, 0)),
            pl.BlockSpec((1, H), lambda i: (0, 0)),
        ],
        out_specs=[
            pl.BlockSpec((1000, NC * R * HW), lambda i: (i, 0)),
            pl.BlockSpec((1000, H), lambda i: (i, 0)),
        ],
        out_shape=[
            jax.ShapeDtypeStruct((N, NC * R * HW), _f32),
            jax.ShapeDtypeStruct((N, H), _f32),
        ],
    )(x, w_rel, w_root, b.reshape(1, H))


def _s_full(s_ref, r):
    return jnp.concatenate(
        [s_ref[c][:, r * HW:(r + 1) * HW] for c in range(NC)], axis=-1)


def _tc_mid_body(root1_ref, s_ref, cnt_ref, wrel_ref, wroot_ref, b_ref,
                 emb_ref, yt_ref, root2_ref):
    cval = cnt_ref[...]
    acc = root1_ref[...]
    for r in range(R):
        d = jnp.maximum(cval[:, r:r + 1] + cval[:, R + r:R + r + 1], 1.0)
        acc = acc + _s_full(s_ref, r) / d
    emb = jnp.maximum(acc, 0.0)
    emb_ref[...] = emb
    ms = [jnp.dot(emb, wrel_ref[r], preferred_element_type=_f32)
          for r in range(R)]
    yt_ref[...] = _pack_yt(ms)
    root2_ref[...] = (
        jnp.dot(emb, wroot_ref[...], preferred_element_type=_f32) + b_ref[...]
    )


def _tc_mid(root1, s_part, cnt8, w_rel, w_root, b):
    return pl.pallas_call(
        _tc_mid_body,
        grid=(10,),
        in_specs=[
            pl.BlockSpec((1000, H), lambda i: (i, 0)),
            pl.BlockSpec((NC, 1000, R * HW), lambda i: (0, i, 0)),
            pl.BlockSpec((1000, 2 * R), lambda i: (i, 0)),
            pl.BlockSpec((R, H, C), lambda i: (0, 0, 0)),
            pl.BlockSpec((H, C), lambda i: (0, 0)),
            pl.BlockSpec((1, C), lambda i: (0, 0)),
        ],
        out_specs=[
            pl.BlockSpec((1000, H), lambda i: (i, 0)),
            pl.BlockSpec((1000, NC * R * HW), lambda i: (i, 0)),
            pl.BlockSpec((1000, C), lambda i: (i, 0)),
        ],
        out_shape=[
            jax.ShapeDtypeStruct((N, H), _f32),
            jax.ShapeDtypeStruct((N, NC * R * HW), _f32),
            jax.ShapeDtypeStruct((N, C), _f32),
        ],
    )(root1, s_part, cnt8, w_rel, w_root, b.reshape(1, C))


def _tc_post_body(root2_ref, s_ref, cnt_ref, out_ref):
    cval = cnt_ref[...]
    logits = root2_ref[...]
    for r in range(R):
        d = jnp.maximum(cval[:, r:r + 1] + cval[:, R + r:R + r + 1], 1.0)
        logits = logits + _s_full(s_ref, r) / d
    m = jnp.max(logits, axis=1, keepdims=True)
    sh = logits - m
    out_ref[...] = sh - jnp.log(jnp.sum(jnp.exp(sh), axis=1, keepdims=True))


def _tc_post(root2, s_part, cnt8):
    return pl.pallas_call(
        _tc_post_body,
        grid=(10,),
        in_specs=[
            pl.BlockSpec((1000, C), lambda i: (i, 0)),
            pl.BlockSpec((NC, 1000, R * HW), lambda i: (0, i, 0)),
            pl.BlockSpec((1000, 2 * R), lambda i: (i, 0)),
        ],
        out_specs=pl.BlockSpec((1000, C), lambda i: (i, 0)),
        out_shape=jax.ShapeDtypeStruct((N, C), _f32),
    )(root2, s_part, cnt8)


_SC_MESH = plsc.VectorSubcoreMesh(core_axis_name="c", subcore_axis_name="s")
_SC_PARAMS = pltpu.CompilerParams(use_tc_tiling_on_sc=False)

_CPT = NCHUNK // NS
_FULL_T = RN // ROWS_PER_TILE
_TAIL = RN - _FULL_T * ROWS_PER_TILE


def _sc_epilogue(cid, sid, tab, out_hbm):
    base = sid * ROWS_PER_TILE

    @pl.when(sid < _FULL_T)
    def _full():
        pltpu.sync_copy(
            tab.at[pl.ds(base, ROWS_PER_TILE)],
            out_hbm.at[cid, pl.ds(base, ROWS_PER_TILE)],
        )

    @pl.when(sid == _FULL_T)
    def _tail():
        pltpu.sync_copy(
            tab.at[pl.ds(_FULL_T * ROWS_PER_TILE, _TAIL)],
            out_hbm.at[cid, pl.ds(_FULL_T * ROWS_PER_TILE, _TAIL)],
        )


def _sc_fill(sid, src_hbm, tab):
    base = sid * ROWS_PER_TILE

    @pl.when(sid < _FULL_T)
    def _full():
        pltpu.sync_copy(src_hbm, tab.at[pl.ds(base, ROWS_PER_TILE)])

    @pl.when(sid == _FULL_T)
    def _tail():
        pltpu.sync_copy(
            src_hbm.at[pl.ds(0, _TAIL)],
            tab.at[pl.ds(_FULL_T * ROWS_PER_TILE, _TAIL)],
        )


def _make_agg(kch, with_count):
    nchunk = SLOTS // kch
    cpt = nchunk // NS

    def body(yt_hbm, epack_hbm, zt_hbm, zc_hbm, ones_hbm,
             s_out, cnt_out, ebuf, gbuf, dbuf, rows, stab,
             gsem, esem, ssem, onesb=None, ctab=None):
        cid = lax.axis_index("c")
        sid = lax.axis_index("s")
        coff = cid * R
        row0 = sid * cpt

        def mk_idx(b):
            for v in range(kch // 16):
                s = ebuf[b, pl.ds(v * 16, 16)]
                d = ebuf[b, pl.ds(kch + v * 16, 16)]
                e = ebuf[b, pl.ds(2 * kch + v * 16, 16)]
                gbuf[b, pl.ds(v * 16, 16)] = s * (NC * R) + e + coff
                dbuf[b, pl.ds(v * 16, 16)] = d * R + e

        _sc_fill(sid, zt_hbm, stab)
        if with_count:
            do_count = jnp.logical_or(
                jnp.logical_and(cid == 0, sid < NS // 2),
                jnp.logical_and(cid == 1, sid >= NS // 2),
            )
            _sc_fill(sid, zc_hbm, ctab)
            pltpu.sync_copy(ones_hbm, onesb)
        plsc.subcore_barrier()

        pltpu.sync_copy(epack_hbm.at[row0], ebuf.at[0])
        mk_idx(0)
        pltpu.async_copy(yt_hbm.at[gbuf.at[0]], rows.at[0], gsem)
        pltpu.async_copy(epack_hbm.at[row0 + 1], ebuf.at[1], esem)

        def pair(jj, carry):
            for b in (0, 1):
                j = jj * 2 + b
                nb = 1 - b

                @pl.when(j > 0)
                def _drain():
                    pltpu.make_async_copy(
                        rows.at[nb], stab.at[dbuf.at[nb]], ssem).wait()

                @pl.when(j < cpt - 1)
                def _prep():
                    pltpu.make_async_copy(
                        epack_hbm.at[row0 + j + 1], ebuf.at[nb], esem).wait()
                    mk_idx(nb)

                pltpu.make_async_copy(
                    yt_hbm.at[gbuf.at[b]], rows.at[b], gsem).wait()

                @pl.when(j < cpt - 1)
                def _gather():
                    pltpu.async_copy(yt_hbm.at[gbuf.at[nb]], rows.at[nb], gsem)

                @pl.when(j < cpt - 2)
                def _edges():
                    pltpu.async_copy(
                        epack_hbm.at[row0 + j + 2], ebuf.at[b], esem)

                pltpu.async_copy(
                    rows.at[b], stab.at[dbuf.at[b]], ssem, add=True)
                if with_count:
                    @pl.when(do_count)
                    def _count():
                        pltpu.sync_copy(onesb, ctab.at[dbuf.at[b]], add=True)
            return carry

        lax.fori_loop(0, cpt // 2, pair, 0)
        last = (cpt - 1) % 2
        pltpu.make_async_copy(
            rows.at[last], stab.at[dbuf.at[last]], ssem).wait()
        plsc.subcore_barrier()
        _sc_epilogue(cid, sid, stab, s_out)
        if with_count:
            _sc_epilogue(cid, sid, ctab, cnt_out)

    scratch = [
        pltpu.VMEM((2, 3 * kch), _i32),
        pltpu.VMEM((2, kch), _i32),
        pltpu.VMEM((2, kch), _i32),
        pltpu.VMEM((2, kch, HW), _f32),
        pltpu.VMEM_SHARED((TROWS, HW), _f32),
        pltpu.SemaphoreType.DMA,
        pltpu.SemaphoreType.DMA,
        pltpu.SemaphoreType.DMA,
    ]
    if with_count:
        scratch += [
            pltpu.VMEM((kch, CW), _f32),
            pltpu.VMEM_SHARED((TROWS, CW), _f32),
        ]
        out_type = [
            jax.ShapeDtypeStruct((NC, RN, HW), _f32),
            jax.ShapeDtypeStruct((NC, RN, CW), _f32),
        ]

        @functools.partial(
            pl.kernel, out_type=out_type, mesh=_SC_MESH,
            compiler_params=_SC_PARAMS, scratch_types=scratch)
        def k(yt_hbm, epack_hbm, zt_hbm, zc_hbm, ones_hbm,
              s_out, cnt_out, *scr):
            body(yt_hbm, epack_hbm, zt_hbm, zc_hbm, ones_hbm,
                 s_out, cnt_out, *scr[:8], onesb=scr[8], ctab=scr[9])
    else:
        @functools.partial(
            pl.kernel, out_type=jax.ShapeDtypeStruct((NC, RN, HW), _f32),
            mesh=_SC_MESH, compiler_params=_SC_PARAMS, scratch_types=scratch)
        def k(yt_hbm, epack_hbm, zt_hbm, zc_hbm, ones_hbm, s_out, *scr):
            body(yt_hbm, epack_hbm, zt_hbm, zc_hbm, ones_hbm,
                 s_out, None, *scr)
    return k


_KCH1 = 256
_KCH2 = 512
_sc_agg_count = _make_agg(_KCH1, True)
_sc_agg = _make_agg(_KCH2, False)


def _pack_edges(srcp, dstp, etp, kch):
    n = SLOTS // kch
    return jnp.stack(
        [srcp.reshape(n, kch), dstp.reshape(n, kch), etp.reshape(n, kch)],
        axis=1,
    ).reshape(n, 3 * kch)


def kernel(x, edge_index, edge_type, W1_rel, W1_root, b1, W2_rel, W2_root, b2):
    src = edge_index[0].astype(_i32)
    dst = edge_index[1].astype(_i32)
    et = edge_type.astype(_i32)
    pad = SLOTS - E
    srcp = jnp.concatenate([src, jnp.zeros((pad,), _i32)])
    dstp = jnp.concatenate([dst, jnp.full((pad,), JUNK_DST, _i32)])
    etp = jnp.concatenate([et, jnp.zeros((pad,), _i32)])
    epack1 = _pack_edges(srcp, dstp, etp, _KCH1)
    epack2 = _pack_edges(srcp, dstp, etp, _KCH2)

    zt = jnp.zeros((ROWS_PER_TILE, HW), _f32)
    zc = jnp.zeros((ROWS_PER_TILE, CW), _f32)
    ones = jnp.ones((_KCH1, CW), _f32)

    yt1, root1 = _tc_pre(x, W1_rel, W1_root, b1)
    s1, cnt_part = _sc_agg_count(
        yt1.reshape(NC * RN, HW), epack1, zt, zc, ones)
    cnt8 = jnp.swapaxes(
        cnt_part[..., 0].reshape(NC, N, R), 0, 1).reshape(N, NC * R)
    s1v = s1.reshape(NC, N, R * HW)

    emb, yt2, root2 = _tc_mid(root1, s1v, cnt8, W2_rel, W2_root, b2)
    s2 = _sc_agg(yt2.reshape(NC * RN, HW), epack2, zt, zc, ones)
    s2v = s2.reshape(NC, N, R * HW)

    logsm = _tc_post(root2, s2v, cnt8)
    return (logsm, emb)
